# baseline (speedup 1.0000x reference)
import functools

import jax
import jax.numpy as jnp
from jax import lax
from jax.experimental import pallas as pl
from jax.experimental.pallas import tpu as pltpu
from jax.experimental.pallas import tpu_sc as plsc

N_VOCAB = 1000000
D_MODEL = 32
BATCH = 4096
HIST = 200

NC = 2
NS = 16
NW = NC * NS
BPW = BATCH // NW
ROWS_PER_SUPER = 128 // D_MODEL
N_SUPER = N_VOCAB // ROWS_PER_SUPER

_mesh = plsc.VectorSubcoreMesh(core_axis_name="c", subcore_axis_name="s")


@functools.partial(
    pl.kernel,
    mesh=_mesh,
    out_type=jax.ShapeDtypeStruct((HIST, D_MODEL, BATCH), jnp.float32),
    scratch_types=[
        pltpu.VMEM((HIST, BPW), jnp.int32),
        pltpu.VMEM((2, BPW), jnp.int32),
        pltpu.VMEM((2, BPW, 128), jnp.float32),
        pltpu.VMEM((2, D_MODEL, BPW), jnp.float32),
        pltpu.SemaphoreType.DMA,
        pltpu.SemaphoreType.DMA,
        pltpu.SemaphoreType.DMA,
        pltpu.SemaphoreType.DMA,
    ],
    compiler_params=pltpu.CompilerParams(
        use_tc_tiling_on_sc=True, needs_layout_passes=False
    ),
)
def _embed_gather(table_hbm, idx_hbm, out_hbm, idx_v, r_v, stage_v, trans_v,
                  gsem0, gsem1, wsem0, wsem1):
    wid = lax.axis_index("s") * NC + lax.axis_index("c")
    b0 = wid * BPW
    gsems = (gsem0, gsem1)
    wsems = (wsem0, wsem1)
    iota16 = lax.iota(jnp.int32, 16)

    pltpu.sync_copy(idx_hbm.at[:, pl.ds(b0, BPW)], idx_v)

    def issue_gather(h, b):
        for l in range(BPW // 16):
            v = idx_v[h, pl.ds(16 * l, 16)]
            r_v[b, pl.ds(16 * l, 16)] = lax.shift_right_logical(v, 2)
        pltpu.async_copy(table_hbm.at[r_v.at[b]], stage_v.at[b], gsems[b])

    def wait_gather(b):
        pltpu.make_async_copy(
            table_hbm.at[pl.ds(0, BPW)], stage_v.at[b], gsems[b]
        ).wait()

    def extract(h, b):
        for l in range(BPW // 16):
            iv = idx_v[h, pl.ds(16 * l, 16)]
            ev = (iv & 3) << 5
            rowi = iota16 + (16 * l)
            for d in range(D_MODEL):
                vals = plsc.load_gather(stage_v.at[b], [rowi, ev + d])
                trans_v[b, d, pl.ds(16 * l, 16)] = vals

    def issue_write(h, b):
        pltpu.async_copy(
            trans_v.at[b], out_hbm.at[h, :, pl.ds(b0, BPW)], wsems[b]
        )

    def wait_write(b):
        pltpu.make_async_copy(
            trans_v.at[b], out_hbm.at[0, :, pl.ds(b0, BPW)], wsems[b]
        ).wait()

    issue_gather(0, 0)
    issue_gather(1, 1)
    wait_gather(0)
    extract(0, 0)
    issue_write(0, 0)

    def pair_body(p, carry):
        for b in range(2):
            h = 2 * p + b
            wait_write(b)
            issue_gather(h, b)
            wait_gather(1 - b)
            extract(h - 1, 1 - b)
            issue_write(h - 1, 1 - b)
        return carry

    lax.fori_loop(1, HIST // 2, pair_body, 0)

    wait_gather(1)
    extract(HIST - 1, 1)
    issue_write(HIST - 1, 1)
    wait_write(0)
    wait_write(1)


def kernel(tok_ids, embedding):
    table4 = embedding.reshape(N_SUPER, 128)
    idx_t = tok_ids.T.astype(jnp.int32)
    out_t = _embed_gather(table4, idx_t)
    return jnp.transpose(out_t, (2, 0, 1))
